# Initial kernel scaffold; baseline (speedup 1.0000x reference)
import jax
import jax.numpy as jnp
from jax.experimental import pallas as pl


def kernel(x, edge_index, edge_attr, batch, W1e, b1e, W2e, b2e, W1n, b1n, W2n, b2n, W1g, b1g, W2g, b2g):
    raise NotImplementedError("write your pallas kernel here")


# speedup vs baseline: 3.1080x; 3.1080x over previous
import functools

import jax
import jax.numpy as jnp
from jax import lax
from jax.experimental import pallas as pl
from jax.experimental.pallas import tpu as pltpu
from jax.experimental.pallas import tpu_sc as plsc

_N, _E, _D, _DE, _HS, _B = 10000, 320000, 128, 16, 128, 8
_NC, _NS = 2, 16
_NW = _NC * _NS
_EPW = _E // _NW
_CE = 80
_NCHUNK = _EPW // _CE
_NPAD = 10240
_RPW = _NPAD // _NS
_ZR = 128
_NBLK = 2000
_LANES = 16


def _sc_mesh():
    return plsc.VectorSubcoreMesh(
        core_axis_name="c", subcore_axis_name="s",
        num_cores=_NC, num_subcores=_NS)


def _gather_add_body(xa_hbm, xb_hbm, src_hbm, dst_hbm, out_hbm,
                     idxa, idxb, bufa, bufb, sema, semb):
    c = lax.axis_index("c")
    s = lax.axis_index("s")
    wid = s * _NC + c
    base = wid * _EPW

    def chunk(i, carry):
        off = base + i * _CE
        pltpu.sync_copy(src_hbm.at[pl.ds(off, _CE)], idxa)
        pltpu.sync_copy(dst_hbm.at[pl.ds(off, _CE)], idxb)
        cpa = pltpu.async_copy(xa_hbm.at[idxa], bufa, sema)
        cpb = pltpu.async_copy(xb_hbm.at[idxb], bufb, semb)
        cpa.wait()
        cpb.wait()

        def row(r, rc):
            for q in range(_HS // _LANES):
                sl = pl.ds(q * _LANES, _LANES)
                bufa[r, sl] = bufa[r, sl] + bufb[r, sl]
            return rc
        lax.fori_loop(0, _CE, row, 0)
        pltpu.sync_copy(bufa, out_hbm.at[pl.ds(off, _CE)])
        return carry

    lax.fori_loop(0, _NCHUNK, chunk, 0)


def _gather_add(xa, xb, src, dst):
    k = functools.partial(
        pl.kernel, mesh=_sc_mesh(),
        out_type=jax.ShapeDtypeStruct((_E, _HS), jnp.float32),
        scratch_types=[
            pltpu.VMEM((_CE,), jnp.int32),
            pltpu.VMEM((_CE,), jnp.int32),
            pltpu.VMEM((_CE, _HS), jnp.float32),
            pltpu.VMEM((_CE, _HS), jnp.float32),
            pltpu.SemaphoreType.DMA,
            pltpu.SemaphoreType.DMA,
        ])(_gather_add_body)
    return k(xa, xb, src, dst)


def _scatter_sum_body(e_hbm, dst_hbm, out_hbm, idxv, bufv, zbuf, acc_sh):
    c = lax.axis_index("c")
    s = lax.axis_index("s")
    wid = s * _NC + c
    zero16 = jnp.zeros((_LANES,), jnp.float32)

    def zrow(r, rc):
        for q in range(_HS // _LANES):
            zbuf[r, pl.ds(q * _LANES, _LANES)] = zero16
        return rc
    lax.fori_loop(0, _ZR, zrow, 0)

    def zchunk(t, tc):
        pltpu.sync_copy(zbuf, acc_sh.at[pl.ds(s * _RPW + t * _ZR, _ZR)])
        return tc
    lax.fori_loop(0, _RPW // _ZR, zchunk, 0)
    plsc.subcore_barrier()

    base = wid * _EPW

    def chunk(i, carry):
        off = base + i * _CE
        pltpu.sync_copy(dst_hbm.at[pl.ds(off, _CE)], idxv)
        pltpu.sync_copy(e_hbm.at[pl.ds(off, _CE)], bufv)
        pltpu.sync_copy(bufv, acc_sh.at[idxv], add=True)
        return carry
    lax.fori_loop(0, _NCHUNK, chunk, 0)
    plsc.subcore_barrier()

    def rchunk(t, tc):
        r0 = s * _RPW + t * _ZR
        pltpu.sync_copy(acc_sh.at[pl.ds(r0, _ZR)],
                        out_hbm.at[pl.ds(c * _NPAD + r0, _ZR)])
        return tc
    lax.fori_loop(0, _RPW // _ZR, rchunk, 0)


def _scatter_sum(e_new, dst):
    k = functools.partial(
        pl.kernel, mesh=_sc_mesh(),
        out_type=jax.ShapeDtypeStruct((_NC * _NPAD, _HS), jnp.float32),
        scratch_types=[
            pltpu.VMEM((_CE,), jnp.int32),
            pltpu.VMEM((_CE, _HS), jnp.float32),
            pltpu.VMEM((_ZR, _HS), jnp.float32),
            pltpu.VMEM_SHARED((_NPAD, _HS), jnp.float32),
        ])(_scatter_sum_body)
    return k(e_new, dst)


def _proj_body(x_ref, wa_ref, wb_ref, xa_ref, xb_ref):
    xblk = x_ref[...]
    xa_ref[...] = jnp.dot(xblk, wa_ref[...], preferred_element_type=jnp.float32)
    xb_ref[...] = jnp.dot(xblk, wb_ref[...], preferred_element_type=jnp.float32)


def _proj(x, wsrc_t, wdst_t):
    return pl.pallas_call(
        _proj_body,
        grid=(_N // _NBLK,),
        in_specs=[
            pl.BlockSpec((_NBLK, _D), lambda i: (i, 0)),
            pl.BlockSpec((_D, _HS), lambda i: (0, 0)),
            pl.BlockSpec((_D, _HS), lambda i: (0, 0)),
        ],
        out_specs=[
            pl.BlockSpec((_NBLK, _HS), lambda i: (i, 0)),
            pl.BlockSpec((_NBLK, _HS), lambda i: (i, 0)),
        ],
        out_shape=[jax.ShapeDtypeStruct((_N, _HS), jnp.float32)] * 2,
    )(x, wsrc_t, wdst_t)


def _edge_mlp_body(s_ref, ea_ref, wat_ref, b1_ref, w2t_ref, b2_ref, out_ref):
    h = (s_ref[...]
         + jnp.dot(ea_ref[...], wat_ref[...], preferred_element_type=jnp.float32)
         + b1_ref[...])
    h = jnp.maximum(h, 0.0)
    out_ref[...] = (jnp.dot(h, w2t_ref[...], preferred_element_type=jnp.float32)
                    + b2_ref[...])


def _edge_mlp(s, edge_attr, wattr_t, b1e, w2e_t, b2e):
    return pl.pallas_call(
        _edge_mlp_body,
        grid=(_E // _NBLK,),
        in_specs=[
            pl.BlockSpec((_NBLK, _HS), lambda i: (i, 0)),
            pl.BlockSpec((_NBLK, _DE), lambda i: (i, 0)),
            pl.BlockSpec((_DE, _HS), lambda i: (0, 0)),
            pl.BlockSpec((1, _HS), lambda i: (0, 0)),
            pl.BlockSpec((_HS, _HS), lambda i: (0, 0)),
            pl.BlockSpec((1, _HS), lambda i: (0, 0)),
        ],
        out_specs=pl.BlockSpec((_NBLK, _HS), lambda i: (i, 0)),
        out_shape=jax.ShapeDtypeStruct((_E, _HS), jnp.float32),
    )(s, edge_attr, wattr_t, b1e, w2e_t, b2e)


def _node_global_body(x_ref, p_ref, b_ref, wx_ref, wg_ref, b1_ref,
                      w2t_ref, b2_ref, w1gt_ref, b1g_ref, w2gt_ref, b2g_ref,
                      xn_ref, u_ref, gacc):
    i = pl.program_id(0)
    agg = p_ref[0] + p_ref[1]
    h = (jnp.dot(x_ref[...], wx_ref[...], preferred_element_type=jnp.float32)
         + jnp.dot(agg, wg_ref[...], preferred_element_type=jnp.float32)
         + b1_ref[...])
    h = jnp.maximum(h, 0.0)
    xn = jnp.dot(h, w2t_ref[...], preferred_element_type=jnp.float32) + b2_ref[...]
    xn_ref[...] = xn
    bids = b_ref[0]
    onehot = (lax.broadcasted_iota(jnp.int32, (_B, _NBLK), 0) == bids
              ).astype(jnp.float32)
    g = jnp.dot(onehot, xn, preferred_element_type=jnp.float32)

    @pl.when(i == 0)
    def _():
        gacc[...] = g

    @pl.when(i > 0)
    def _():
        gacc[...] = gacc[...] + g

    @pl.when(i == pl.num_programs(0) - 1)
    def _():
        gg = jnp.maximum(
            jnp.dot(gacc[...], w1gt_ref[...], preferred_element_type=jnp.float32)
            + b1g_ref[...], 0.0)
        u_ref[...] = (jnp.dot(gg, w2gt_ref[...], preferred_element_type=jnp.float32)
                      + b2g_ref[...])


def _node_global(x, parts, batch3d, wx_t, wagg_t, b1n, w2n_t, b2n,
                 w1g_t, b1g, w2g_t, b2g):
    nb = _N // _NBLK
    return pl.pallas_call(
        _node_global_body,
        grid=(nb,),
        in_specs=[
            pl.BlockSpec((_NBLK, _D), lambda i: (i, 0)),
            pl.BlockSpec((2, _NBLK, _HS), lambda i: (0, i, 0)),
            pl.BlockSpec((1, 1, _NBLK), lambda i: (i, 0, 0)),
            pl.BlockSpec((_D, _HS), lambda i: (0, 0)),
            pl.BlockSpec((_HS, _HS), lambda i: (0, 0)),
            pl.BlockSpec((1, _HS), lambda i: (0, 0)),
            pl.BlockSpec((_HS, _HS), lambda i: (0, 0)),
            pl.BlockSpec((1, _HS), lambda i: (0, 0)),
            pl.BlockSpec((_HS, _HS), lambda i: (0, 0)),
            pl.BlockSpec((1, _HS), lambda i: (0, 0)),
            pl.BlockSpec((_HS, _HS), lambda i: (0, 0)),
            pl.BlockSpec((1, _HS), lambda i: (0, 0)),
        ],
        out_specs=[
            pl.BlockSpec((_NBLK, _HS), lambda i: (i, 0)),
            pl.BlockSpec((_B, _HS), lambda i: (0, 0)),
        ],
        out_shape=[
            jax.ShapeDtypeStruct((_N, _HS), jnp.float32),
            jax.ShapeDtypeStruct((_B, _HS), jnp.float32),
        ],
        scratch_shapes=[pltpu.VMEM((_B, _HS), jnp.float32)],
    )(x, parts, batch3d, wx_t, wagg_t, b1n, w2n_t, b2n, w1g_t, b1g, w2g_t, b2g)


def kernel(x, edge_index, edge_attr, batch,
           W1e, b1e, W2e, b2e,
           W1n, b1n, W2n, b2n,
           W1g, b1g, W2g, b2g):
    src = edge_index[0]
    dst = edge_index[1]
    wsrc_t = W1e[:, :_D].T
    wdst_t = W1e[:, _D:2 * _D].T
    wattr_t = W1e[:, 2 * _D:].T

    xa, xb = _proj(x, wsrc_t, wdst_t)
    s = _gather_add(xa, xb, src, dst)
    e_new = _edge_mlp(s, edge_attr, wattr_t, b1e.reshape(1, _HS),
                      W2e.T, b2e.reshape(1, _HS))
    parts = _scatter_sum(e_new, dst).reshape(_NC, _NPAD, _HS)[:, :_N, :]

    batch3d = batch.reshape(_N // _NBLK, 1, _NBLK)
    x_new, u = _node_global(
        x, parts, batch3d,
        W1n[:, :_D].T, W1n[:, _D:].T, b1n.reshape(1, _HS),
        W2n.T, b2n.reshape(1, _HS),
        W1g.T, b1g.reshape(1, _HS), W2g.T, b2g.reshape(1, _HS))
    return (x_new, e_new, u)


# speedup vs baseline: 4.8845x; 1.5716x over previous
import functools

import jax
import jax.numpy as jnp
from jax import lax
from jax.experimental import pallas as pl
from jax.experimental.pallas import tpu as pltpu
from jax.experimental.pallas import tpu_sc as plsc

_N, _E, _D, _DE, _HS, _B = 10000, 320000, 128, 16, 128, 8
_NC, _NS = 2, 16
_NW = _NC * _NS
_EPW = _E // _NW
_CE = 80
_NCHUNK = _EPW // _CE
_NPAD = 10240
_RPW = _NPAD // _NS
_ZR = 128
_NBLK = 2000
_LANES = 16


def _sc_mesh():
    return plsc.VectorSubcoreMesh(
        core_axis_name="c", subcore_axis_name="s",
        num_cores=_NC, num_subcores=_NS)


def _gather_add_body(xa_hbm, xb_hbm, src_hbm, dst_hbm, out_hbm,
                     idxa, idxb,
                     bufa0, bufa1, bufb0, bufb1, bufo0, bufo1,
                     ga0, ga1, gb0, gb1, w0, w1):
    c = lax.axis_index("c")
    s = lax.axis_index("s")
    wid = s * _NC + c
    base = wid * _EPW
    bufa = (bufa0, bufa1)
    bufb = (bufb0, bufb1)
    bufo = (bufo0, bufo1)
    gsa = (ga0, ga1)
    gsb = (gb0, gb1)
    wsm = (w0, w1)

    pltpu.sync_copy(src_hbm.at[wid], idxa)
    pltpu.sync_copy(dst_hbm.at[wid], idxb)

    def issue(i, slot):
        pltpu.async_copy(xa_hbm.at[idxa.at[i]], bufa[slot], gsa[slot])
        pltpu.async_copy(xb_hbm.at[idxb.at[i]], bufb[slot], gsb[slot])

    def process(i, slot, first, last):
        pltpu.make_async_copy(xa_hbm.at[idxa.at[i]], bufa[slot], gsa[slot]).wait()
        pltpu.make_async_copy(xb_hbm.at[idxb.at[i]], bufb[slot], gsb[slot]).wait()
        if not first:
            pltpu.make_async_copy(
                bufo[slot], out_hbm.at[pl.ds(base, _CE)], wsm[slot]).wait()

        def row(r, rc):
            for q in range(_HS // _LANES):
                sl = pl.ds(q * _LANES, _LANES)
                bufo[slot][r, sl] = bufa[slot][r, sl] + bufb[slot][r, sl]
            return rc
        lax.fori_loop(0, _CE, row, 0)

        @pl.when(i + 2 < _NCHUNK)
        def _():
            issue(i + 2, slot)
        if last:
            pltpu.sync_copy(bufo[slot], out_hbm.at[pl.ds(base + i * _CE, _CE)])
        else:
            pltpu.async_copy(bufo[slot],
                             out_hbm.at[pl.ds(base + i * _CE, _CE)], wsm[slot])

    issue(0, 0)
    issue(1, 1)

    def pair(g, carry):
        process(2 * g, 0, False, False)
        process(2 * g + 1, 1, False, False)
        return carry

    process(0, 0, True, False)
    process(1, 1, True, False)
    lax.fori_loop(1, (_NCHUNK - 1) // 2, pair, 0)
    process(_NCHUNK - 1, 0, False, True)
    pltpu.make_async_copy(bufo[1], out_hbm.at[pl.ds(base, _CE)], wsm[1]).wait()


def _gather_add(xa, xb, src3, dst3):
    dmasem = pltpu.SemaphoreType.DMA
    k = functools.partial(
        pl.kernel, mesh=_sc_mesh(),
        out_type=jax.ShapeDtypeStruct((_E, _HS), jnp.float32),
        scratch_types=[
            pltpu.VMEM((_NCHUNK, _CE), jnp.int32),
            pltpu.VMEM((_NCHUNK, _CE), jnp.int32),
            pltpu.VMEM((_CE, _HS), jnp.float32),
            pltpu.VMEM((_CE, _HS), jnp.float32),
            pltpu.VMEM((_CE, _HS), jnp.float32),
            pltpu.VMEM((_CE, _HS), jnp.float32),
            pltpu.VMEM((_CE, _HS), jnp.float32),
            pltpu.VMEM((_CE, _HS), jnp.float32),
            dmasem, dmasem, dmasem, dmasem, dmasem, dmasem,
        ])(_gather_add_body)
    return k(xa, xb, src3, dst3)


def _scatter_sum_body(e_hbm, dst_hbm, out_hbm, idxv, buf0, buf1, acc_sh,
                      f0, f1):
    c = lax.axis_index("c")
    s = lax.axis_index("s")
    wid = s * _NC + c
    bufv = (buf0, buf1)
    fsm = (f0, f1)
    zero16 = jnp.zeros((_LANES,), jnp.float32)

    def zrow(r, rc):
        for q in range(_HS // _LANES):
            buf0[r, pl.ds(q * _LANES, _LANES)] = zero16
        return rc
    lax.fori_loop(0, _CE, zrow, 0)

    def zchunk(t, tc):
        pltpu.sync_copy(buf0, acc_sh.at[pl.ds(s * _RPW + t * _CE, _CE)])
        return tc
    lax.fori_loop(0, _RPW // _CE, zchunk, 0)
    pltpu.sync_copy(dst_hbm.at[wid], idxv)
    plsc.subcore_barrier()

    base = wid * _EPW

    def issue(i, slot):
        pltpu.async_copy(e_hbm.at[pl.ds(base + i * _CE, _CE)],
                         bufv[slot], fsm[slot])

    def process(i, slot):
        pltpu.make_async_copy(e_hbm.at[pl.ds(base, _CE)],
                              bufv[slot], fsm[slot]).wait()
        pltpu.sync_copy(bufv[slot], acc_sh.at[idxv.at[i]], add=True)

        @pl.when(i + 2 < _NCHUNK)
        def _():
            issue(i + 2, slot)

    issue(0, 0)
    issue(1, 1)

    def pair(g, carry):
        process(2 * g, 0)
        process(2 * g + 1, 1)
        return carry

    lax.fori_loop(0, (_NCHUNK - 1) // 2, pair, 0)
    process(_NCHUNK - 1, 0)
    plsc.subcore_barrier()

    def rchunk(t, tc):
        r0 = s * _RPW + t * _CE
        pltpu.sync_copy(acc_sh.at[pl.ds(r0, _CE)],
                        out_hbm.at[pl.ds(c * _NPAD + r0, _CE)])
        return tc
    lax.fori_loop(0, _RPW // _CE, rchunk, 0)


def _scatter_sum(e_new, dst3):
    dmasem = pltpu.SemaphoreType.DMA
    k = functools.partial(
        pl.kernel, mesh=_sc_mesh(),
        out_type=jax.ShapeDtypeStruct((_NC * _NPAD, _HS), jnp.float32),
        scratch_types=[
            pltpu.VMEM((_NCHUNK, _CE), jnp.int32),
            pltpu.VMEM((_CE, _HS), jnp.float32),
            pltpu.VMEM((_CE, _HS), jnp.float32),
            pltpu.VMEM_SHARED((_NPAD, _HS), jnp.float32),
            dmasem, dmasem,
        ])(_scatter_sum_body)
    return k(e_new, dst3)


def _proj_body(x_ref, wa_ref, wb_ref, xa_ref, xb_ref):
    xblk = x_ref[...]
    xa_ref[...] = jnp.dot(xblk, wa_ref[...], preferred_element_type=jnp.float32)
    xb_ref[...] = jnp.dot(xblk, wb_ref[...], preferred_element_type=jnp.float32)


def _proj(x, wsrc_t, wdst_t):
    return pl.pallas_call(
        _proj_body,
        grid=(_N // _NBLK,),
        in_specs=[
            pl.BlockSpec((_NBLK, _D), lambda i: (i, 0)),
            pl.BlockSpec((_D, _HS), lambda i: (0, 0)),
            pl.BlockSpec((_D, _HS), lambda i: (0, 0)),
        ],
        out_specs=[
            pl.BlockSpec((_NBLK, _HS), lambda i: (i, 0)),
            pl.BlockSpec((_NBLK, _HS), lambda i: (i, 0)),
        ],
        out_shape=[jax.ShapeDtypeStruct((_N, _HS), jnp.float32)] * 2,
    )(x, wsrc_t, wdst_t)


def _edge_mlp_body(s_ref, ea_ref, wat_ref, b1_ref, w2t_ref, b2_ref, out_ref):
    h = (s_ref[...]
         + jnp.dot(ea_ref[...], wat_ref[...], preferred_element_type=jnp.float32)
         + b1_ref[...])
    h = jnp.maximum(h, 0.0)
    out_ref[...] = (jnp.dot(h, w2t_ref[...], preferred_element_type=jnp.float32)
                    + b2_ref[...])


def _edge_mlp(s, edge_attr, wattr_t, b1e, w2e_t, b2e):
    return pl.pallas_call(
        _edge_mlp_body,
        grid=(_E // _NBLK,),
        in_specs=[
            pl.BlockSpec((_NBLK, _HS), lambda i: (i, 0)),
            pl.BlockSpec((_NBLK, _DE), lambda i: (i, 0)),
            pl.BlockSpec((_DE, _HS), lambda i: (0, 0)),
            pl.BlockSpec((1, _HS), lambda i: (0, 0)),
            pl.BlockSpec((_HS, _HS), lambda i: (0, 0)),
            pl.BlockSpec((1, _HS), lambda i: (0, 0)),
        ],
        out_specs=pl.BlockSpec((_NBLK, _HS), lambda i: (i, 0)),
        out_shape=jax.ShapeDtypeStruct((_E, _HS), jnp.float32),
    )(s, edge_attr, wattr_t, b1e, w2e_t, b2e)


def _node_global_body(x_ref, p_ref, b_ref, wx_ref, wg_ref, b1_ref,
                      w2t_ref, b2_ref, w1gt_ref, b1g_ref, w2gt_ref, b2g_ref,
                      xn_ref, u_ref, gacc):
    i = pl.program_id(0)
    agg = p_ref[0] + p_ref[1]
    h = (jnp.dot(x_ref[...], wx_ref[...], preferred_element_type=jnp.float32)
         + jnp.dot(agg, wg_ref[...], preferred_element_type=jnp.float32)
         + b1_ref[...])
    h = jnp.maximum(h, 0.0)
    xn = jnp.dot(h, w2t_ref[...], preferred_element_type=jnp.float32) + b2_ref[...]
    xn_ref[...] = xn
    bids = b_ref[0]
    onehot = (lax.broadcasted_iota(jnp.int32, (_B, _NBLK), 0) == bids
              ).astype(jnp.float32)
    g = jnp.dot(onehot, xn, preferred_element_type=jnp.float32)

    @pl.when(i == 0)
    def _():
        gacc[...] = g

    @pl.when(i > 0)
    def _():
        gacc[...] = gacc[...] + g

    @pl.when(i == pl.num_programs(0) - 1)
    def _():
        gg = jnp.maximum(
            jnp.dot(gacc[...], w1gt_ref[...], preferred_element_type=jnp.float32)
            + b1g_ref[...], 0.0)
        u_ref[...] = (jnp.dot(gg, w2gt_ref[...], preferred_element_type=jnp.float32)
                      + b2g_ref[...])


def _node_global(x, parts, batch3d, wx_t, wagg_t, b1n, w2n_t, b2n,
                 w1g_t, b1g, w2g_t, b2g):
    nb = _N // _NBLK
    return pl.pallas_call(
        _node_global_body,
        grid=(nb,),
        in_specs=[
            pl.BlockSpec((_NBLK, _D), lambda i: (i, 0)),
            pl.BlockSpec((2, _NBLK, _HS), lambda i: (0, i, 0)),
            pl.BlockSpec((1, 1, _NBLK), lambda i: (i, 0, 0)),
            pl.BlockSpec((_D, _HS), lambda i: (0, 0)),
            pl.BlockSpec((_HS, _HS), lambda i: (0, 0)),
            pl.BlockSpec((1, _HS), lambda i: (0, 0)),
            pl.BlockSpec((_HS, _HS), lambda i: (0, 0)),
            pl.BlockSpec((1, _HS), lambda i: (0, 0)),
            pl.BlockSpec((_HS, _HS), lambda i: (0, 0)),
            pl.BlockSpec((1, _HS), lambda i: (0, 0)),
            pl.BlockSpec((_HS, _HS), lambda i: (0, 0)),
            pl.BlockSpec((1, _HS), lambda i: (0, 0)),
        ],
        out_specs=[
            pl.BlockSpec((_NBLK, _HS), lambda i: (i, 0)),
            pl.BlockSpec((_B, _HS), lambda i: (0, 0)),
        ],
        out_shape=[
            jax.ShapeDtypeStruct((_N, _HS), jnp.float32),
            jax.ShapeDtypeStruct((_B, _HS), jnp.float32),
        ],
        scratch_shapes=[pltpu.VMEM((_B, _HS), jnp.float32)],
    )(x, parts, batch3d, wx_t, wagg_t, b1n, w2n_t, b2n, w1g_t, b1g, w2g_t, b2g)


def kernel(x, edge_index, edge_attr, batch,
           W1e, b1e, W2e, b2e,
           W1n, b1n, W2n, b2n,
           W1g, b1g, W2g, b2g):
    src3 = edge_index[0].reshape(_NW, _NCHUNK, _CE)
    dst3 = edge_index[1].reshape(_NW, _NCHUNK, _CE)
    wsrc_t = W1e[:, :_D].T
    wdst_t = W1e[:, _D:2 * _D].T
    wattr_t = W1e[:, 2 * _D:].T

    xa, xb = _proj(x, wsrc_t, wdst_t)
    s = _gather_add(xa, xb, src3, dst3)
    e_new = _edge_mlp(s, edge_attr, wattr_t, b1e.reshape(1, _HS),
                      W2e.T, b2e.reshape(1, _HS))
    parts = _scatter_sum(e_new, dst3).reshape(_NC, _NPAD, _HS)[:, :_N, :]

    batch3d = batch.reshape(_N // _NBLK, 1, _NBLK)
    x_new, u = _node_global(
        x, parts, batch3d,
        W1n[:, :_D].T, W1n[:, _D:].T, b1n.reshape(1, _HS),
        W2n.T, b2n.reshape(1, _HS),
        W1g.T, b1g.reshape(1, _HS), W2g.T, b2g.reshape(1, _HS))
    return (x_new, e_new, u)


# speedup vs baseline: 4.8846x; 1.0000x over previous
import functools

import jax
import jax.numpy as jnp
from jax import lax
from jax.experimental import pallas as pl
from jax.experimental.pallas import tpu as pltpu
from jax.experimental.pallas import tpu_sc as plsc

_N, _E, _D, _DE, _HS, _B = 10000, 320000, 128, 16, 128, 8
_NC, _NS = 2, 16
_NW = _NC * _NS
_EPW = _E // _NW
_CE = 80
_NCHUNK = _EPW // _CE
_NPAD = 10240
_RPW = _NPAD // _NS
_ZR = 128
_NBLK = 2000
_LANES = 16


def _sc_mesh():
    return plsc.VectorSubcoreMesh(
        core_axis_name="c", subcore_axis_name="s",
        num_cores=_NC, num_subcores=_NS)


def _gather_add_body(xa_hbm, xb_hbm, src_hbm, dst_hbm, out_hbm,
                     idxa, idxb,
                     bufa0, bufa1, bufb0, bufb1, bufo0, bufo1,
                     ga0, ga1, gb0, gb1, w0, w1):
    c = lax.axis_index("c")
    s = lax.axis_index("s")
    wid = s * _NC + c
    base = wid * _EPW
    bufa = (bufa0, bufa1)
    bufb = (bufb0, bufb1)
    bufo = (bufo0, bufo1)
    gsa = (ga0, ga1)
    gsb = (gb0, gb1)
    wsm = (w0, w1)

    pltpu.sync_copy(src_hbm.at[wid], idxa)
    pltpu.sync_copy(dst_hbm.at[wid], idxb)

    def issue(i, slot):
        pltpu.async_copy(xa_hbm.at[idxa.at[i]], bufa[slot], gsa[slot])
        pltpu.async_copy(xb_hbm.at[idxb.at[i]], bufb[slot], gsb[slot])

    def process(i, slot, first, last):
        pltpu.make_async_copy(xa_hbm.at[idxa.at[i]], bufa[slot], gsa[slot]).wait()
        pltpu.make_async_copy(xb_hbm.at[idxb.at[i]], bufb[slot], gsb[slot]).wait()
        if not first:
            pltpu.make_async_copy(
                bufo[slot], out_hbm.at[pl.ds(base, _CE)], wsm[slot]).wait()

        def row(r, rc):
            for q in range(_HS // _LANES):
                sl = pl.ds(q * _LANES, _LANES)
                bufo[slot][r, sl] = bufa[slot][r, sl] + bufb[slot][r, sl]
            return rc
        lax.fori_loop(0, _CE, row, 0)

        @pl.when(i + 2 < _NCHUNK)
        def _():
            issue(i + 2, slot)
        if last:
            pltpu.sync_copy(bufo[slot], out_hbm.at[pl.ds(base + i * _CE, _CE)])
        else:
            pltpu.async_copy(bufo[slot],
                             out_hbm.at[pl.ds(base + i * _CE, _CE)], wsm[slot])

    issue(0, 0)
    issue(1, 1)

    def pair(g, carry):
        process(2 * g, 0, False, False)
        process(2 * g + 1, 1, False, False)
        return carry

    process(0, 0, True, False)
    process(1, 1, True, False)
    lax.fori_loop(1, (_NCHUNK - 1) // 2, pair, 0)
    process(_NCHUNK - 1, 0, False, True)
    pltpu.make_async_copy(bufo[1], out_hbm.at[pl.ds(base, _CE)], wsm[1]).wait()


def _gather_add(xa, xb, src3, dst3):
    dmasem = pltpu.SemaphoreType.DMA
    k = functools.partial(
        pl.kernel, mesh=_sc_mesh(),
        out_type=jax.ShapeDtypeStruct((_E, _HS), jnp.float32),
        scratch_types=[
            pltpu.VMEM((_NCHUNK, _CE), jnp.int32),
            pltpu.VMEM((_NCHUNK, _CE), jnp.int32),
            pltpu.VMEM((_CE, _HS), jnp.float32),
            pltpu.VMEM((_CE, _HS), jnp.float32),
            pltpu.VMEM((_CE, _HS), jnp.float32),
            pltpu.VMEM((_CE, _HS), jnp.float32),
            pltpu.VMEM((_CE, _HS), jnp.float32),
            pltpu.VMEM((_CE, _HS), jnp.float32),
            dmasem, dmasem, dmasem, dmasem, dmasem, dmasem,
        ])(_gather_add_body)
    return k(xa, xb, src3, dst3)


def _scatter_sum_body(e_hbm, dst_hbm, out_hbm, idxv, buf0, buf1, acc_sh,
                      f0, f1):
    c = lax.axis_index("c")
    s = lax.axis_index("s")
    wid = s * _NC + c
    bufv = (buf0, buf1)
    fsm = (f0, f1)
    zero16 = jnp.zeros((_LANES,), jnp.float32)

    def zrow(r, rc):
        for q in range(_HS // _LANES):
            buf0[r, pl.ds(q * _LANES, _LANES)] = zero16
        return rc
    lax.fori_loop(0, _CE, zrow, 0)

    def zchunk(t, tc):
        pltpu.sync_copy(buf0, acc_sh.at[pl.ds(s * _RPW + t * _CE, _CE)])
        return tc
    lax.fori_loop(0, _RPW // _CE, zchunk, 0)
    pltpu.sync_copy(dst_hbm.at[wid], idxv)
    plsc.subcore_barrier()

    base = wid * _EPW

    def issue(i, slot):
        pltpu.async_copy(e_hbm.at[pl.ds(base + i * _CE, _CE)],
                         bufv[slot], fsm[slot])

    def process(i, slot):
        pltpu.make_async_copy(e_hbm.at[pl.ds(base, _CE)],
                              bufv[slot], fsm[slot]).wait()
        pltpu.sync_copy(bufv[slot], acc_sh.at[idxv.at[i]], add=True)

        @pl.when(i + 2 < _NCHUNK)
        def _():
            issue(i + 2, slot)

    issue(0, 0)
    issue(1, 1)

    def pair(g, carry):
        process(2 * g, 0)
        process(2 * g + 1, 1)
        return carry

    lax.fori_loop(0, (_NCHUNK - 1) // 2, pair, 0)
    process(_NCHUNK - 1, 0)
    plsc.subcore_barrier()

    def rchunk(t, tc):
        r0 = s * _RPW + t * _CE
        pltpu.sync_copy(acc_sh.at[pl.ds(r0, _CE)],
                        out_hbm.at[pl.ds(c * _NPAD + r0, _CE)])
        return tc
    lax.fori_loop(0, _RPW // _CE, rchunk, 0)


def _scatter_sum(e_new, dst3):
    dmasem = pltpu.SemaphoreType.DMA
    k = functools.partial(
        pl.kernel, mesh=_sc_mesh(),
        out_type=jax.ShapeDtypeStruct((_NC * _NPAD, _HS), jnp.float32),
        scratch_types=[
            pltpu.VMEM((_NCHUNK, _CE), jnp.int32),
            pltpu.VMEM((_CE, _HS), jnp.float32),
            pltpu.VMEM((_CE, _HS), jnp.float32),
            pltpu.VMEM_SHARED((_NPAD, _HS), jnp.float32),
            dmasem, dmasem,
        ])(_scatter_sum_body)
    return k(e_new, dst3)


def _proj_body(x_ref, wa_ref, wb_ref, xa_ref, xb_ref):
    xblk = x_ref[...]
    xa_ref[...] = jnp.dot(xblk, wa_ref[...], preferred_element_type=jnp.float32)
    xb_ref[...] = jnp.dot(xblk, wb_ref[...], preferred_element_type=jnp.float32)


def _proj(x, wsrc_t, wdst_t):
    return pl.pallas_call(
        _proj_body,
        grid=(_N // _NBLK,),
        in_specs=[
            pl.BlockSpec((_NBLK, _D), lambda i: (i, 0)),
            pl.BlockSpec((_D, _HS), lambda i: (0, 0)),
            pl.BlockSpec((_D, _HS), lambda i: (0, 0)),
        ],
        out_specs=[
            pl.BlockSpec((_NBLK, _HS), lambda i: (i, 0)),
            pl.BlockSpec((_NBLK, _HS), lambda i: (i, 0)),
        ],
        out_shape=[jax.ShapeDtypeStruct((_N, _HS), jnp.float32)] * 2,
    )(x, wsrc_t, wdst_t)


def _edge_mlp_body(s_ref, ea_ref, wat_ref, b1_ref, w2t_ref, b2_ref, out_ref):
    h = (s_ref[...]
         + jnp.dot(ea_ref[...], wat_ref[...], preferred_element_type=jnp.float32)
         + b1_ref[...])
    h = jnp.maximum(h, 0.0)
    out_ref[...] = (jnp.dot(h, w2t_ref[...], preferred_element_type=jnp.float32)
                    + b2_ref[...])


def _edge_mlp(s, edge_attr, wattr_t, b1e, w2e_t, b2e):
    return pl.pallas_call(
        _edge_mlp_body,
        grid=(_E // _NBLK,),
        in_specs=[
            pl.BlockSpec((_NBLK, _HS), lambda i: (i, 0)),
            pl.BlockSpec((_NBLK, _DE), lambda i: (i, 0)),
            pl.BlockSpec((_DE, _HS), lambda i: (0, 0)),
            pl.BlockSpec((1, _HS), lambda i: (0, 0)),
            pl.BlockSpec((_HS, _HS), lambda i: (0, 0)),
            pl.BlockSpec((1, _HS), lambda i: (0, 0)),
        ],
        out_specs=pl.BlockSpec((_NBLK, _HS), lambda i: (i, 0)),
        out_shape=jax.ShapeDtypeStruct((_E, _HS), jnp.float32),
    )(s, edge_attr, wattr_t, b1e, w2e_t, b2e)


def _node_global_body(x_ref, p_ref, b_ref, wx_ref, wg_ref, b1_ref,
                      w2t_ref, b2_ref, w1gt_ref, b1g_ref, w2gt_ref, b2g_ref,
                      xn_ref, u_ref, gacc):
    i = pl.program_id(0)
    agg = p_ref[0] + p_ref[1]
    h = (jnp.dot(x_ref[...], wx_ref[...], preferred_element_type=jnp.float32)
         + jnp.dot(agg, wg_ref[...], preferred_element_type=jnp.float32)
         + b1_ref[...])
    h = jnp.maximum(h, 0.0)
    xn = jnp.dot(h, w2t_ref[...], preferred_element_type=jnp.float32) + b2_ref[...]
    xn_ref[...] = xn
    bids = b_ref[0]
    onehot = (lax.broadcasted_iota(jnp.int32, (_B, _NBLK), 0) == bids
              ).astype(jnp.float32)
    g = jnp.dot(onehot, xn, preferred_element_type=jnp.float32)

    @pl.when(i == 0)
    def _():
        gacc[...] = g

    @pl.when(i > 0)
    def _():
        gacc[...] = gacc[...] + g

    @pl.when(i == pl.num_programs(0) - 1)
    def _():
        gg = jnp.maximum(
            jnp.dot(gacc[...], w1gt_ref[...], preferred_element_type=jnp.float32)
            + b1g_ref[...], 0.0)
        u_ref[...] = (jnp.dot(gg, w2gt_ref[...], preferred_element_type=jnp.float32)
                      + b2g_ref[...])


def _node_global(x, parts, batch3d, wx_t, wagg_t, b1n, w2n_t, b2n,
                 w1g_t, b1g, w2g_t, b2g):
    nb = _N // _NBLK
    return pl.pallas_call(
        _node_global_body,
        grid=(nb,),
        in_specs=[
            pl.BlockSpec((_NBLK, _D), lambda i: (i, 0)),
            pl.BlockSpec((2, _NBLK, _HS), lambda i: (0, i, 0)),
            pl.BlockSpec((1, 1, _NBLK), lambda i: (i, 0, 0)),
            pl.BlockSpec((_D, _HS), lambda i: (0, 0)),
            pl.BlockSpec((_HS, _HS), lambda i: (0, 0)),
            pl.BlockSpec((1, _HS), lambda i: (0, 0)),
            pl.BlockSpec((_HS, _HS), lambda i: (0, 0)),
            pl.BlockSpec((1, _HS), lambda i: (0, 0)),
            pl.BlockSpec((_HS, _HS), lambda i: (0, 0)),
            pl.BlockSpec((1, _HS), lambda i: (0, 0)),
            pl.BlockSpec((_HS, _HS), lambda i: (0, 0)),
            pl.BlockSpec((1, _HS), lambda i: (0, 0)),
        ],
        out_specs=[
            pl.BlockSpec((_NBLK, _HS), lambda i: (i, 0)),
            pl.BlockSpec((_B, _HS), lambda i: (0, 0)),
        ],
        out_shape=[
            jax.ShapeDtypeStruct((_N, _HS), jnp.float32),
            jax.ShapeDtypeStruct((_B, _HS), jnp.float32),
        ],
        scratch_shapes=[pltpu.VMEM((_B, _HS), jnp.float32)],
    )(x, parts, batch3d, wx_t, wagg_t, b1n, w2n_t, b2n, w1g_t, b1g, w2g_t, b2g)


def kernel(x, edge_index, edge_attr, batch,
           W1e, b1e, W2e, b2e,
           W1n, b1n, W2n, b2n,
           W1g, b1g, W2g, b2g):
    src3 = edge_index[0].reshape(_NW, _NCHUNK, _CE)
    dst3 = edge_index[1].reshape(_NW, _NCHUNK, _CE)
    wsrc_t = W1e[:, :_D].T
    wdst_t = W1e[:, _D:2 * _D].T
    wattr_t = W1e[:, 2 * _D:].T

    xa_f, xb_f = _proj(x, wsrc_t, wdst_t)
    s = _gather_add(xa_f, xb_f, src3, dst3)
    e_new = _edge_mlp(s, edge_attr, wattr_t, b1e.reshape(1, _HS),
                      W2e.T, b2e.reshape(1, _HS))
    parts = _scatter_sum(e_new, dst3).reshape(_NC, _NPAD, _HS)[:, :_N, :]

    batch3d = batch.reshape(_N // _NBLK, 1, _NBLK)
    x_new, u = _node_global(
        x, parts, batch3d,
        W1n[:, :_D].T, W1n[:, _D:].T, b1n.reshape(1, _HS),
        W2n.T, b2n.reshape(1, _HS),
        W1g.T, b1g.reshape(1, _HS), W2g.T, b2g.reshape(1, _HS))
    return (x_new, e_new, u)
